# baseline (speedup 1.0000x reference)
import jax
import jax.numpy as jnp
from jax.experimental import pallas as pl

INPUT_DIM = 2048
NUM_EXPERTS = 16
TOP_K = 2
NUM_TOKENS = 16384

BLK = 2048
NSTEP = NUM_TOKENS // BLK


def _gating_kernel(x_ref, wt_ref, b_ref, w_out_ref, i_out_ref):
    logits = jnp.dot(x_ref[...], wt_ref[...], preferred_element_type=jnp.float32)
    logits = logits + b_ref[...]
    lanes = jax.lax.broadcasted_iota(jnp.int32, logits.shape, 1)
    l1 = jnp.max(logits, axis=1, keepdims=True)
    i1 = jnp.min(jnp.where(logits == l1, lanes, NUM_EXPERTS), axis=1, keepdims=True)
    l_masked = jnp.where(lanes == i1, -jnp.inf, logits)
    l2 = jnp.max(l_masked, axis=1, keepdims=True)
    i2 = jnp.min(
        jnp.where(l_masked == l2, lanes, NUM_EXPERTS), axis=1, keepdims=True
    )
    s = jnp.sum(jnp.exp(logits - l1), axis=1, keepdims=True)
    w2 = jnp.exp(l2 - l1)
    w_out_ref[...] = jnp.concatenate([jnp.ones_like(w2), w2], axis=1) / s
    i_out_ref[...] = jnp.concatenate([i1, i2], axis=1)


@jax.jit
def kernel(x, W, b):
    wt = W.T
    b2 = b.reshape(1, NUM_EXPERTS)
    w_out, i_out = pl.pallas_call(
        _gating_kernel,
        grid=(NSTEP,),
        in_specs=[
            pl.BlockSpec((BLK, INPUT_DIM), lambda i: (i, 0)),
            pl.BlockSpec((INPUT_DIM, NUM_EXPERTS), lambda i: (0, 0)),
            pl.BlockSpec((1, NUM_EXPERTS), lambda i: (0, 0)),
        ],
        out_specs=[
            pl.BlockSpec((BLK, TOP_K), lambda i: (i, 0)),
            pl.BlockSpec((BLK, TOP_K), lambda i: (i, 0)),
        ],
        out_shape=[
            jax.ShapeDtypeStruct((NUM_TOKENS, TOP_K), jnp.float32),
            jax.ShapeDtypeStruct((NUM_TOKENS, TOP_K), jnp.int32),
        ],
    )(x, wt, b2)
    return (w_out, i_out)
